# baseline (speedup 1.0000x reference)
import jax
import jax.numpy as jnp
import numpy as np
from jax.experimental import pallas as pl
from jax.experimental.pallas import tpu as pltpu


def _router_kernel(x_ref, wt_ref, bsw_ref, out_ref, acc_ref):
    t = pl.program_id(0)
    nt = pl.num_programs(0)
    B, ST, D = x_ref.shape

    @pl.when(t == 0)
    def _init():
        acc_ref[...] = jnp.zeros_like(acc_ref)

    wt = wt_ref[...]
    for b in range(B):
        al = [jnp.zeros((8, D), jnp.float32) for _ in range(4)]
        for g in range(ST // 8):
            xt = x_ref[b, 8 * g:8 * g + 8, :]
            for ss in range(8):
                s = 8 * g + ss
                al[s % 4] = al[s % 4] + wt[:, D * s:D * (s + 1)] * xt[ss:ss + 1, :]
        acc_ref[8 * b:8 * b + 8, :] += ((al[0] + al[1]) + (al[2] + al[3]))

    @pl.when(t == nt - 1)
    def _fin():
        accT = jnp.transpose(acc_ref[...])
        s = jnp.sum(accT, axis=0, keepdims=True)
        lg = s + bsw_ref[0:1, 0:8 * B]
        L = jnp.broadcast_to(lg, (8, 8 * B))
        lane = jax.lax.broadcasted_iota(jnp.int32, (8, 8 * B), 1)
        neg = jnp.float32(-jnp.inf)
        tops = []
        for b in range(B):
            inb = (lane >= 8 * b) & (lane < 8 * b + 8)
            vals = jnp.where(inb, L, neg)
            m1 = jnp.max(vals, axis=1, keepdims=True)
            i1 = jnp.min(jnp.where(vals == m1, lane, 127),
                         axis=1, keepdims=True)
            vals2 = jnp.where(lane == i1, neg, vals)
            m2 = jnp.max(vals2, axis=1, keepdims=True)
            i2 = jnp.min(jnp.where(vals2 == m2, lane, 127),
                         axis=1, keepdims=True)
            tops.append((i1 - 8 * b, i2 - 8 * b))
        row = jax.lax.broadcasted_iota(jnp.int32, (8, 128), 0)
        lane_o = jax.lax.broadcasted_iota(jnp.int32, (8, 128), 1)
        i1a, i2a = tops[-1]
        for b in range(B - 2, -1, -1):
            i1a = jnp.where(row == b, tops[b][0], i1a)
            i2a = jnp.where(row == b, tops[b][1], i2a)
        out_ref[...] = jnp.where(lane_o == 0, i1a,
                                 jnp.where(lane_o == 1, i2a, 0)).astype(jnp.int32)


def _route(x, W_switch, b_switch):
    B, S, D = x.shape
    SD = S * D
    wt = W_switch.T
    bsw = jnp.tile(b_switch.astype(jnp.float32), (8, 16))
    nt = max(1, min(32, S // 8))
    ST = S // nt
    C = ST * D
    topmat = pl.pallas_call(
        _router_kernel,
        grid=(nt,),
        in_specs=[
            pl.BlockSpec((B, ST, D), lambda t: (0, t, 0)),
            pl.BlockSpec((8, C), lambda t: (0, t)),
            pl.BlockSpec((8, 128), lambda t: (0, 0)),
        ],
        out_specs=pl.BlockSpec((8, 128), lambda t: (0, 0)),
        out_shape=jax.ShapeDtypeStruct((8, 128), jnp.int32),
        scratch_shapes=[pltpu.VMEM((8 * B, D), jnp.float32)],
    )(x, wt, bsw)
    return topmat[:B, :2]


def _ffn_kernel(idx_ref, x_ref, w1_ref, b1_ref, w2_ref, b2_ref, out_ref):
    k = pl.program_id(2)
    xb = x_ref[0]
    h = jnp.dot(xb, w1_ref[0], preferred_element_type=jnp.float32)
    h = h + b1_ref[0]
    h = 0.5 * h * (1.0 + jax.lax.erf(h * np.float32(0.7071067811865476)))
    o = jnp.dot(h, w2_ref[0], preferred_element_type=jnp.float32)
    o = o + b2_ref[0]

    @pl.when(k == 0)
    def _store():
        out_ref[0] = o

    @pl.when(k != 0)
    def _acc():
        out_ref[0] += o


def kernel(x, W_switch, b_switch, W1, b1, W2, b2):
    B, S, D = x.shape
    E, _, SUBH = W1.shape
    K = 2

    topi = _route(x, W_switch, b_switch)
    idx = topi.reshape(B * K)

    b1r = b1.reshape(E, 1, SUBH)
    b2r = b2.reshape(E, 1, D)

    ST = min(S, 1024)
    grid_spec = pltpu.PrefetchScalarGridSpec(
        num_scalar_prefetch=1,
        grid=(B, S // ST, K),
        in_specs=[
            pl.BlockSpec((1, ST, D), lambda b, s, k, idx: (b, s, 0)),
            pl.BlockSpec((1, D, SUBH),
                         lambda b, s, k, idx: (idx[b * 2 + k], 0, 0)),
            pl.BlockSpec((1, 1, SUBH),
                         lambda b, s, k, idx: (idx[b * 2 + k], 0, 0)),
            pl.BlockSpec((1, SUBH, D),
                         lambda b, s, k, idx: (idx[b * 2 + k], 0, 0)),
            pl.BlockSpec((1, 1, D),
                         lambda b, s, k, idx: (idx[b * 2 + k], 0, 0)),
        ],
        out_specs=pl.BlockSpec((1, ST, D), lambda b, s, k, idx: (b, s, 0)),
    )
    out = pl.pallas_call(
        _ffn_kernel,
        grid_spec=grid_spec,
        out_shape=jax.ShapeDtypeStruct((B, S, D), jnp.float32),
    )(idx, x, W1, b1r, W2, b2r)
    return out


# speedup vs baseline: 1.1386x; 1.1386x over previous
import jax
import jax.numpy as jnp
import numpy as np
from jax.experimental import pallas as pl
from jax.experimental.pallas import tpu as pltpu


def _router_kernel(x_ref, wt_ref, bsw_ref, out_ref, acc_ref):
    t = pl.program_id(0)
    nt = pl.num_programs(0)
    B, ST, D = x_ref.shape

    @pl.when(t == 0)
    def _init():
        acc_ref[...] = jnp.zeros_like(acc_ref)

    wt = wt_ref[...]
    for b in range(B):
        al = [jnp.zeros((8, D), jnp.float32) for _ in range(4)]
        for g in range(ST // 8):
            xt = x_ref[b, 8 * g:8 * g + 8, :]
            for ss in range(8):
                s = 8 * g + ss
                al[s % 4] = al[s % 4] + wt[:, D * s:D * (s + 1)] * xt[ss:ss + 1, :]
        acc_ref[8 * b:8 * b + 8, :] += ((al[0] + al[1]) + (al[2] + al[3]))

    @pl.when(t == nt - 1)
    def _fin():
        accT = jnp.transpose(acc_ref[...])
        s = jnp.sum(accT, axis=0, keepdims=True)
        lg = s + bsw_ref[0:1, 0:8 * B]
        L = jnp.broadcast_to(lg, (8, 8 * B))
        lane = jax.lax.broadcasted_iota(jnp.int32, (8, 8 * B), 1)
        neg = jnp.float32(-jnp.inf)
        tops = []
        for b in range(B):
            inb = (lane >= 8 * b) & (lane < 8 * b + 8)
            vals = jnp.where(inb, L, neg)
            m1 = jnp.max(vals, axis=1, keepdims=True)
            i1 = jnp.min(jnp.where(vals == m1, lane, 127),
                         axis=1, keepdims=True)
            vals2 = jnp.where(lane == i1, neg, vals)
            m2 = jnp.max(vals2, axis=1, keepdims=True)
            i2 = jnp.min(jnp.where(vals2 == m2, lane, 127),
                         axis=1, keepdims=True)
            tops.append((i1 - 8 * b, i2 - 8 * b))
        lane_o = jax.lax.broadcasted_iota(jnp.int32, (8, 128), 1)
        vals = jnp.zeros((8, 128), jnp.float32)
        for b in range(B):
            vals = jnp.where(lane_o == 2 * b, tops[b][0].astype(jnp.float32),
                             vals)
            vals = jnp.where(lane_o == 2 * b + 1,
                             tops[b][1].astype(jnp.float32), vals)
        out_ref[...] = vals.astype(jnp.int32)


def _route(x, W_switch, b_switch):
    B, S, D = x.shape
    SD = S * D
    wt = W_switch.T
    bsw = jnp.tile(b_switch.astype(jnp.float32), (8, 16))
    nt = max(1, min(16, S // 8))
    ST = S // nt
    C = ST * D
    topmat = pl.pallas_call(
        _router_kernel,
        grid=(nt,),
        in_specs=[
            pl.BlockSpec((B, ST, D), lambda t: (0, t, 0)),
            pl.BlockSpec((8, C), lambda t: (0, t)),
            pl.BlockSpec((8, 128), lambda t: (0, 0)),
        ],
        out_specs=pl.BlockSpec((8, 128), lambda t: (0, 0)),
        out_shape=jax.ShapeDtypeStruct((8, 128), jnp.int32),
        scratch_shapes=[pltpu.VMEM((8 * B, D), jnp.float32)],
    )(x, wt, bsw)
    return topmat


def _ffn_kernel(idx_ref, x_ref, w1_ref, b1_ref, w2_ref, b2_ref, out_ref):
    b = pl.program_id(0)
    k = pl.program_id(2)
    e = idx_ref[0, b * 2 + k]
    xb = x_ref[0].astype(jnp.bfloat16)
    h = jnp.dot(xb, w1_ref[0].astype(jnp.bfloat16),
                preferred_element_type=jnp.float32)
    h = h + b1_ref[pl.ds(e, 1), :]
    h = 0.5 * h * (1.0 + jax.lax.erf(h * np.float32(0.7071067811865476)))
    o = jnp.dot(h.astype(jnp.bfloat16), w2_ref[0].astype(jnp.bfloat16),
                preferred_element_type=jnp.float32)
    o = o + b2_ref[pl.ds(e, 1), :]

    @pl.when(k == 0)
    def _store():
        out_ref[0] = o

    @pl.when(k != 0)
    def _acc():
        out_ref[0] += o


def kernel(x, W_switch, b_switch, W1, b1, W2, b2):
    B, S, D = x.shape
    E, _, SUBH = W1.shape
    K = 2

    topmat = _route(x, W_switch, b_switch)

    ST = min(S, 1024)
    grid_spec = pltpu.PrefetchScalarGridSpec(
        num_scalar_prefetch=1,
        grid=(B, S // ST, K),
        in_specs=[
            pl.BlockSpec((1, ST, D), lambda b, s, k, idx: (b, s, 0)),
            pl.BlockSpec((1, D, SUBH),
                         lambda b, s, k, idx: (idx[0, b * 2 + k], 0, 0)),
            pl.BlockSpec((E, SUBH), lambda b, s, k, idx: (0, 0)),
            pl.BlockSpec((1, SUBH, D),
                         lambda b, s, k, idx: (idx[0, b * 2 + k], 0, 0)),
            pl.BlockSpec((E, D), lambda b, s, k, idx: (0, 0)),
        ],
        out_specs=pl.BlockSpec((1, ST, D), lambda b, s, k, idx: (b, s, 0)),
    )
    out = pl.pallas_call(
        _ffn_kernel,
        grid_spec=grid_spec,
        out_shape=jax.ShapeDtypeStruct((B, S, D), jnp.float32),
    )(topmat, x, W1, b1, W2, b2)
    return out


# speedup vs baseline: 1.1853x; 1.0410x over previous
import jax
import jax.numpy as jnp
import numpy as np
from jax.experimental import pallas as pl
from jax.experimental.pallas import tpu as pltpu


def _router_kernel(x_ref, wt_ref, bsw_ref, out_ref, acc_ref):
    t = pl.program_id(0)
    nt = pl.num_programs(0)
    B, ST, D = x_ref.shape

    @pl.when(t == 0)
    def _init():
        acc_ref[...] = jnp.zeros_like(acc_ref)

    wt = wt_ref[...]
    for b in range(B):
        al = [jnp.zeros((8, D), jnp.float32) for _ in range(4)]
        for g in range(ST // 8):
            xt = x_ref[b, 8 * g:8 * g + 8, :]
            for ss in range(8):
                s = 8 * g + ss
                al[s % 4] = al[s % 4] + wt[:, D * s:D * (s + 1)] * xt[ss:ss + 1, :]
        acc_ref[8 * b:8 * b + 8, :] += ((al[0] + al[1]) + (al[2] + al[3]))

    @pl.when(t == nt - 1)
    def _fin():
        accT = jnp.transpose(acc_ref[...])
        s = jnp.sum(accT, axis=0, keepdims=True)
        lg = s + bsw_ref[0:1, 0:8 * B]
        L = jnp.broadcast_to(lg, (8, 8 * B))
        lane = jax.lax.broadcasted_iota(jnp.int32, (8, 8 * B), 1)
        neg = jnp.float32(-jnp.inf)
        tops = []
        for b in range(B):
            inb = (lane >= 8 * b) & (lane < 8 * b + 8)
            vals = jnp.where(inb, L, neg)
            m1 = jnp.max(vals, axis=1, keepdims=True)
            i1 = jnp.min(jnp.where(vals == m1, lane, 127),
                         axis=1, keepdims=True)
            vals2 = jnp.where(lane == i1, neg, vals)
            m2 = jnp.max(vals2, axis=1, keepdims=True)
            i2 = jnp.min(jnp.where(vals2 == m2, lane, 127),
                         axis=1, keepdims=True)
            tops.append((i1 - 8 * b, i2 - 8 * b))
        lane_o = jax.lax.broadcasted_iota(jnp.int32, (8, 128), 1)
        vals = jnp.zeros((8, 128), jnp.float32)
        for b in range(B):
            vals = jnp.where(lane_o == 2 * b, tops[b][0].astype(jnp.float32),
                             vals)
            vals = jnp.where(lane_o == 2 * b + 1,
                             tops[b][1].astype(jnp.float32), vals)
        out_ref[...] = vals.astype(jnp.int32)


def _route(x, W_switch, b_switch):
    B, S, D = x.shape
    SD = S * D
    wt = W_switch.T
    bsw = jnp.tile(b_switch.astype(jnp.float32), (8, 16))
    nt = max(1, min(8, S // 8))
    ST = S // nt
    C = ST * D
    topmat = pl.pallas_call(
        _router_kernel,
        grid=(nt,),
        in_specs=[
            pl.BlockSpec((B, ST, D), lambda t: (0, t, 0)),
            pl.BlockSpec((8, C), lambda t: (0, t)),
            pl.BlockSpec((8, 128), lambda t: (0, 0)),
        ],
        out_specs=pl.BlockSpec((8, 128), lambda t: (0, 0)),
        out_shape=jax.ShapeDtypeStruct((8, 128), jnp.int32),
        scratch_shapes=[pltpu.VMEM((8 * B, D), jnp.float32)],
    )(x, wt, bsw)
    return topmat


def _ffn_kernel(idx_ref, x_ref, w1_ref, b1_ref, w2_ref, b2_ref, out_ref):
    b = pl.program_id(0)
    k = pl.program_id(2)
    e = idx_ref[0, b * 2 + k]
    xb = x_ref[0].astype(jnp.bfloat16)
    h = jnp.dot(xb, w1_ref[0].astype(jnp.bfloat16),
                preferred_element_type=jnp.float32)
    h = h + b1_ref[pl.ds(e, 1), :]
    h = 0.5 * h * (1.0 + jax.lax.erf(h * np.float32(0.7071067811865476)))
    o = jnp.dot(h.astype(jnp.bfloat16), w2_ref[0].astype(jnp.bfloat16),
                preferred_element_type=jnp.float32)
    o = o + b2_ref[pl.ds(e, 1), :]

    @pl.when(k == 0)
    def _store():
        out_ref[0] = o

    @pl.when(k != 0)
    def _acc():
        out_ref[0] += o


def kernel(x, W_switch, b_switch, W1, b1, W2, b2):
    B, S, D = x.shape
    E, _, SUBH = W1.shape
    K = 2

    topmat = _route(x, W_switch, b_switch)

    ST = min(S, 1024)
    grid_spec = pltpu.PrefetchScalarGridSpec(
        num_scalar_prefetch=1,
        grid=(B, S // ST, K),
        in_specs=[
            pl.BlockSpec((1, ST, D), lambda b, s, k, idx: (b, s, 0)),
            pl.BlockSpec((1, D, SUBH),
                         lambda b, s, k, idx: (idx[0, b * 2 + k], 0, 0)),
            pl.BlockSpec((E, SUBH), lambda b, s, k, idx: (0, 0)),
            pl.BlockSpec((1, SUBH, D),
                         lambda b, s, k, idx: (idx[0, b * 2 + k], 0, 0)),
            pl.BlockSpec((E, D), lambda b, s, k, idx: (0, 0)),
        ],
        out_specs=pl.BlockSpec((1, ST, D), lambda b, s, k, idx: (b, s, 0)),
    )
    out = pl.pallas_call(
        _ffn_kernel,
        grid_spec=grid_spec,
        out_shape=jax.ShapeDtypeStruct((B, S, D), jnp.float32),
    )(topmat, x, W1, b1, W2, b2)
    return out
